# baseline (speedup 1.0000x reference)
import functools

import jax
import jax.numpy as jnp
from jax import lax
from jax.experimental import pallas as pl
from jax.experimental.pallas import tpu as pltpu
from jax.experimental.pallas import tpu_sc as plsc

EMB_NUM = 1000000
EMB_DIM = 64
CHUNK = 128
NBUF = 4
NW = 32
TC_BLK = 1024


def _pad_rows(weight_t):
    V = weight_t.shape[1]
    grid = (V + TC_BLK - 1) // TC_BLK

    def body(in_ref, out_ref):
        x = in_ref[...]
        eye = jnp.float32(
            lax.broadcasted_iota(jnp.int32, (EMB_DIM, EMB_DIM), 0)
            == lax.broadcasted_iota(jnp.int32, (EMB_DIM, EMB_DIM), 1))
        xh = x.astype(jnp.bfloat16).astype(jnp.float32)
        xl = x - xh
        dn = (((0,), (0,)), ((), ()))
        out_ref[:, 0:EMB_DIM] = (
            lax.dot_general(xh, eye, dn, preferred_element_type=jnp.float32)
            + lax.dot_general(xl, eye, dn, preferred_element_type=jnp.float32))

    return pl.pallas_call(
        body,
        grid=(grid,),
        in_specs=[pl.BlockSpec((EMB_DIM, TC_BLK), lambda i: (0, i))],
        out_specs=pl.BlockSpec((TC_BLK, 128), lambda i: (i, 0)),
        out_shape=jax.ShapeDtypeStruct((V, 128), jnp.float32),
    )(weight_t)


def _to_feature_major(rows, hist, batch):
    nb = batch // TC_BLK

    def body(in_ref, out_ref):
        y = in_ref[...]
        sel = jnp.float32(
            lax.broadcasted_iota(jnp.int32, (EMB_DIM, 128), 0)
            == lax.broadcasted_iota(jnp.int32, (EMB_DIM, 128), 1))
        yh = y.astype(jnp.bfloat16).astype(jnp.float32)
        yl = y - yh
        dn = (((1,), (1,)), ((), ()))
        out_ref[0, :, :] = (
            lax.dot_general(sel, yh, dn, preferred_element_type=jnp.float32)
            + lax.dot_general(sel, yl, dn, preferred_element_type=jnp.float32))

    return pl.pallas_call(
        body,
        grid=(hist, nb),
        in_specs=[pl.BlockSpec((TC_BLK, 128), lambda h, i: (h * nb + i, 0))],
        out_specs=pl.BlockSpec((1, EMB_DIM, TC_BLK), lambda h, i: (h, 0, i)),
        out_shape=jax.ShapeDtypeStruct((hist, EMB_DIM, batch), jnp.float32),
    )(rows)


@functools.lru_cache(maxsize=None)
def _build_gather(batch: int, hist: int):
    b_per_w = batch // NW
    bc_per_h = b_per_w // CHUNK
    n_chunks = bc_per_h * hist
    assert (n_chunks - NBUF) % NBUF == 0
    mesh = plsc.VectorSubcoreMesh(core_axis_name="c", subcore_axis_name="s")

    @functools.partial(
        pl.kernel,
        mesh=mesh,
        compiler_params=pltpu.CompilerParams(use_tc_tiling_on_sc=True),
        out_type=jax.ShapeDtypeStruct((hist * batch, 128), jnp.float32),
        scratch_types=[
            pltpu.VMEM((n_chunks, CHUNK), jnp.int32),
            *[pltpu.VMEM((CHUNK, 128), jnp.float32) for _ in range(NBUF)],
            *[pltpu.SemaphoreType.DMA for _ in range(2 * NBUF)],
        ],
    )
    def kern(idx_hbm, table_hbm, out_hbm, idx_v, *bufs_and_sems):
        rows = bufs_and_sems[:NBUF]
        sem_g = bufs_and_sems[NBUF:2 * NBUF]
        sem_s = bufs_and_sems[2 * NBUF:]
        wid = lax.axis_index("s") * 2 + lax.axis_index("c")
        base_b = wid * b_per_w
        pltpu.sync_copy(idx_hbm.at[pl.ds(wid * n_chunks, n_chunks)], idx_v)

        def out_slice(c):
            r0 = (c // bc_per_h) * batch + base_b + (c % bc_per_h) * CHUNK
            return out_hbm.at[pl.ds(r0, CHUNK)]

        def start_gather(c, b):
            pltpu.async_copy(table_hbm.at[idx_v.at[c]], rows[b], sem_g[b])

        def wait_gather(b):
            pltpu.make_async_copy(table_hbm.at[idx_v.at[0]], rows[b],
                                  sem_g[b]).wait()

        def start_store(c, b):
            pltpu.async_copy(rows[b], out_slice(c), sem_s[b])

        def wait_store(b):
            pltpu.make_async_copy(rows[b], out_hbm.at[pl.ds(0, CHUNK)],
                                  sem_s[b]).wait()

        start_gather(0, 0)
        for g in range(NBUF - 1):
            start_gather(g + 1, (g + 1) % NBUF)
            wait_gather(g % NBUF)
            start_store(g, g % NBUF)

        def body(i, carry):
            for j in range(NBUF):
                g = (NBUF - 1) + i * NBUF + j
                b = (NBUF - 1 + j) % NBUF
                bn = (b + 1) % NBUF
                wait_store(bn)
                start_gather(g + 1, bn)
                wait_gather(b)
                start_store(g, b)
            return carry

        lax.fori_loop(0, (n_chunks - NBUF) // NBUF, body, 0)

        g_last = n_chunks - 1
        wait_gather(g_last % NBUF)
        start_store(g_last, g_last % NBUF)
        for b in range(NBUF):
            wait_store(b)

    return kern


def kernel(inputs, weight):
    batch, hist = inputs.shape
    table = _pad_rows(weight.T)
    idx = (inputs.T.astype(jnp.int32)
           .reshape(hist, NW, batch // NW // CHUNK, CHUNK)
           .transpose(1, 0, 2, 3)
           .reshape(-1, CHUNK))
    rows = _build_gather(batch, hist)(idx, table)
    out = _to_feature_major(rows, hist, batch)
    return lax.transpose(out, (2, 0, 1))


# speedup vs baseline: 1.3582x; 1.3582x over previous
import functools

import jax
import jax.numpy as jnp
from jax import lax
from jax.experimental import pallas as pl
from jax.experimental.pallas import tpu as pltpu
from jax.experimental.pallas import tpu_sc as plsc

EMB_DIM = 64
IDXW = 100
BE_CHUNK = 8
NBUF = 4
NW = 32


@functools.lru_cache(maxsize=None)
def _build(batch: int, hist: int):
    be_per_w = batch // NW
    rows_per_chunk = BE_CHUNK * hist
    k = rows_per_chunk // IDXW
    n_chunks = be_per_w // BE_CHUNK
    rows_per_w = be_per_w * hist
    assert (n_chunks - NBUF) % NBUF == 0 and n_chunks > 2 * NBUF
    mesh = plsc.VectorSubcoreMesh(core_axis_name="c", subcore_axis_name="s")

    @functools.partial(
        pl.kernel,
        mesh=mesh,
        compiler_params=pltpu.CompilerParams(use_tc_tiling_on_sc=False),
        out_type=jax.ShapeDtypeStruct((batch, hist, EMB_DIM), jnp.float32),
        scratch_types=[
            pltpu.VMEM((rows_per_w // IDXW, IDXW), jnp.int32),
            *[pltpu.VMEM((BE_CHUNK * hist, EMB_DIM), jnp.float32)
              for _ in range(NBUF)],
            *[pltpu.SemaphoreType.DMA for _ in range(2 * NBUF)],
        ],
    )
    def kern(idx_hbm, table_hbm, out_hbm, idx_v, *bufs_and_sems):
        rows = bufs_and_sems[:NBUF]
        sem_g = bufs_and_sems[NBUF:2 * NBUF]
        sem_s = bufs_and_sems[2 * NBUF:]
        wid = lax.axis_index("s") * 2 + lax.axis_index("c")
        base_be = wid * be_per_w
        pltpu.sync_copy(idx_hbm.at[pl.ds(wid * (rows_per_w // IDXW),
                                         rows_per_w // IDXW)], idx_v)

        def start_gather(c, b):
            for q in range(k):
                pltpu.async_copy(
                    table_hbm.at[idx_v.at[c * k + q]],
                    rows[b].at[pl.ds(q * IDXW, IDXW)],
                    sem_g[b],
                )

        def wait_gather(b):
            for q in range(k):
                pltpu.make_async_copy(
                    table_hbm.at[idx_v.at[q]],
                    rows[b].at[pl.ds(q * IDXW, IDXW)],
                    sem_g[b],
                ).wait()

        def start_store(c, b):
            for e in range(BE_CHUNK):
                pltpu.async_copy(
                    rows[b].at[pl.ds(e * hist, hist)],
                    out_hbm.at[base_be + c * BE_CHUNK + e],
                    sem_s[b])

        def wait_store(b):
            for e in range(BE_CHUNK):
                pltpu.make_async_copy(
                    rows[b].at[pl.ds(e * hist, hist)],
                    out_hbm.at[0], sem_s[b]).wait()

        start_gather(0, 0)
        for g in range(NBUF - 1):
            start_gather(g + 1, (g + 1) % NBUF)
            wait_gather(g % NBUF)
            start_store(g, g % NBUF)

        def body(i, carry):
            for j in range(NBUF):
                g = (NBUF - 1) + i * NBUF + j
                b = (NBUF - 1 + j) % NBUF
                bn = (b + 1) % NBUF
                wait_store(bn)
                start_gather(g + 1, bn)
                wait_gather(b)
                start_store(g, b)
            return carry

        lax.fori_loop(0, (n_chunks - NBUF) // NBUF, body, 0)

        g_last = n_chunks - 1
        wait_gather(g_last % NBUF)
        start_store(g_last, g_last % NBUF)
        for b in range(NBUF):
            wait_store(b)

    return kern


def kernel(inputs, weight):
    batch, hist = inputs.shape
    idx = inputs.astype(jnp.int32).reshape(-1, IDXW)
    return _build(batch, hist)(idx, weight)


# speedup vs baseline: 1.3622x; 1.0029x over previous
import functools

import jax
import jax.numpy as jnp
from jax import lax
from jax.experimental import pallas as pl
from jax.experimental.pallas import tpu as pltpu
from jax.experimental.pallas import tpu_sc as plsc

EMB_DIM = 64
CHUNK = 128
GROUP = 256
NBUF = 4
NW = 32
K = GROUP // CHUNK


@functools.lru_cache(maxsize=None)
def _build(B: int):
    b_per_w = B // NW
    n_groups = b_per_w // GROUP
    assert n_groups % NBUF == 0 and n_groups > 2 * NBUF
    mesh = plsc.VectorSubcoreMesh(core_axis_name="c", subcore_axis_name="s")

    @functools.partial(
        pl.kernel,
        mesh=mesh,
        compiler_params=pltpu.CompilerParams(use_tc_tiling_on_sc=False),
        out_type=jax.ShapeDtypeStruct((B, EMB_DIM), jnp.float32),
        scratch_types=[
            pltpu.VMEM((b_per_w // CHUNK, CHUNK), jnp.int32),
            *[pltpu.VMEM((GROUP, EMB_DIM), jnp.float32) for _ in range(NBUF)],
            *[pltpu.SemaphoreType.DMA for _ in range(2 * NBUF)],
        ],
    )
    def kern(idx_hbm, table_hbm, out_hbm, idx_v, *bufs_and_sems):
        rows = bufs_and_sems[:NBUF]
        sem_g = bufs_and_sems[NBUF:2 * NBUF]
        sem_s = bufs_and_sems[2 * NBUF:]
        wid = lax.axis_index("s") * 2 + lax.axis_index("c")
        base = wid * b_per_w
        pltpu.sync_copy(idx_hbm.at[pl.ds(wid * (b_per_w // CHUNK),
                                         b_per_w // CHUNK)], idx_v)

        def start_gather(g, b):
            for c in range(K):
                pltpu.async_copy(
                    table_hbm.at[idx_v.at[g * K + c]],
                    rows[b].at[pl.ds(c * CHUNK, CHUNK)],
                    sem_g[b],
                )

        def wait_gather(b):
            for c in range(K):
                pltpu.make_async_copy(
                    table_hbm.at[idx_v.at[c]],
                    rows[b].at[pl.ds(c * CHUNK, CHUNK)],
                    sem_g[b],
                ).wait()

        def start_store(g, b):
            pltpu.async_copy(rows[b], out_hbm.at[pl.ds(base + g * GROUP, GROUP)],
                             sem_s[b])

        def wait_store(b):
            pltpu.make_async_copy(rows[b], out_hbm.at[pl.ds(base, GROUP)],
                                  sem_s[b]).wait()

        start_gather(0, 0)
        for g in range(NBUF - 1):
            start_gather(g + 1, (g + 1) % NBUF)
            wait_gather(g % NBUF)
            start_store(g, g % NBUF)

        def body(i, carry):
            for j in range(NBUF):
                g = (NBUF - 1) + i * NBUF + j
                b = (NBUF - 1 + j) % NBUF
                bn = (b + 1) % NBUF
                wait_store(bn)
                start_gather(g + 1, bn)
                wait_gather(b)
                start_store(g, b)
            return carry

        lax.fori_loop(0, (n_groups - NBUF) // NBUF, body, 0)

        g_last = n_groups - 1
        wait_gather(g_last % NBUF)
        start_store(g_last, g_last % NBUF)
        for b in range(NBUF):
            wait_store(b)

    return kern


def kernel(inputs, weight):
    batch, hist = inputs.shape
    B = batch * hist
    idx = inputs.reshape(B // CHUNK, CHUNK).astype(jnp.int32)
    out = _build(B)(idx, weight)
    return out.reshape(batch, hist, EMB_DIM)
